# baseline (speedup 1.0000x reference)
import functools

import jax
import jax.numpy as jnp
from jax import lax
from jax.experimental import pallas as pl
from jax.experimental.pallas import tpu as pltpu
from jax.experimental.pallas import tpu_sc as plsc

N = 10000
M = 2000
NNZ = 160000
D = 512

NC, NS = 2, 16
NW = NC * NS
EPT = 5120
E_PAD = EPT * NW
ME = 2048
NV = 10240
RPT_E = ME // NS
RPT_V = NV // NS

KA, NCHA = 128, 40
CBA = 128
NCBA = D // CBA
KB, NCHB = 128, 40
CBB = 128
NCBB = D // CBB
REP = 8

_mesh = plsc.VectorSubcoreMesh(core_axis_name="c", subcore_axis_name="s")
_sc_params = pltpu.CompilerParams(needs_layout_passes=False)


def _zero_vmem_2d(ref, rows, cols):
    def body(r, _):
        for l in range(cols // 16):
            ref[r, pl.ds(l * 16, 16)] = jnp.zeros((16,), jnp.float32)
        return 0
    lax.fori_loop(0, rows, body, 0)


def _zero_vmem_1d(ref, n):
    def body(l, _):
        ref[pl.ds(l * 16, 16)] = jnp.zeros((16,), jnp.float32)
        return 0
    lax.fori_loop(0, n // 16, body, 0)


def _gs_pipeline(src, gidx, dst, sidx, stags, sems, nch):
    nbuf = len(stags)
    ngrp = nch // nbuf
    for b in range(nbuf - 1):
        pltpu.async_copy(src.at[gidx.at[b]], stags[b], sems[b])

    def group(g, _):
        j0 = g * nbuf
        for b in range(nbuf):
            j = j0 + b
            pltpu.make_async_copy(src.at[gidx.at[j]], stags[b], sems[b]).wait()
            nb = (b + nbuf - 1) % nbuf
            pltpu.async_copy(src.at[gidx.at[j + nbuf - 1]], stags[nb], sems[nb])
            pltpu.sync_copy(stags[b], dst.at[sidx.at[j]], add=True)
        return 0

    lax.fori_loop(0, ngrp - 1, group, 0)
    j0 = (ngrp - 1) * nbuf
    for b in range(nbuf):
        j = j0 + b
        pltpu.make_async_copy(src.at[gidx.at[j]], stags[b], sems[b]).wait()
        if j + nbuf - 1 < nch:
            nb = (b + nbuf - 1) % nbuf
            pltpu.async_copy(src.at[gidx.at[j + nbuf - 1]], stags[nb], sems[nb])
        pltpu.sync_copy(stags[b], dst.at[sidx.at[j]], add=True)


@functools.partial(
    pl.kernel,
    out_type=(
        jax.ShapeDtypeStruct((NC, ME, D), jnp.float32),
        jax.ShapeDtypeStruct((NW, ME), jnp.float32),
        jax.ShapeDtypeStruct((NW, NV), jnp.float32),
    ),
    mesh=_mesh,
    scratch_types=[
        pltpu.VMEM((NCHA, KA), jnp.int32),
        pltpu.VMEM((NCHA, KA), jnp.int32),
        pltpu.VMEM((KA, CBA), jnp.float32),
        pltpu.VMEM((KA, CBA), jnp.float32),
        pltpu.VMEM((KA, CBA), jnp.float32),
        pltpu.VMEM((KA, CBA), jnp.float32),
        pltpu.VMEM((ME,), jnp.float32),
        pltpu.VMEM((NV,), jnp.float32),
        pltpu.VMEM_SHARED((ME, CBA), jnp.float32),
        pltpu.SemaphoreType.DMA,
        pltpu.SemaphoreType.DMA,
        pltpu.SemaphoreType.DMA,
        pltpu.SemaphoreType.DMA,
    ],
    compiler_params=_sc_params,
)
def _phase_a(x0c, x1c, x2c, x3c, v_hbm, e_hbm, xep, cep, cvp,
             vidx, eidx, stag0, stag1, stag2, stag3, ceh, cvh, xes,
             sem0, sem1, sem2, sem3):
    cid = lax.axis_index("c")
    sid = lax.axis_index("s")
    wid = cid * NS + sid

    pltpu.sync_copy(v_hbm.at[wid], vidx)
    pltpu.sync_copy(e_hbm.at[wid], eidx)

    _zero_vmem_1d(ceh, ME)
    _zero_vmem_1d(cvh, NV)

    ones = jnp.ones((16,), jnp.float32)

    def hist(j, _):
        for l in range(KA // 16):
            vv = vidx[j, pl.ds(l * 16, 16)]
            ev = eidx[j, pl.ds(l * 16, 16)]
            plsc.addupdate_scatter(cvh, [vv], ones)
            plsc.addupdate_scatter(ceh, [ev], ones)
        return 0

    lax.fori_loop(0, NCHA, hist, 0)
    pltpu.sync_copy(ceh, cep.at[wid])
    pltpu.sync_copy(cvh, cvp.at[wid])

    for cb, xc in enumerate((x0c, x1c, x2c, x3c)):
        _zero_vmem_2d(stag0, KA, CBA)
        pltpu.sync_copy(stag0, xes.at[pl.ds(sid * RPT_E, RPT_E)])
        plsc.subcore_barrier()

        _gs_pipeline(xc, vidx, xes, eidx,
                     (stag0, stag1, stag2, stag3),
                     (sem0, sem1, sem2, sem3), NCHA)
        plsc.subcore_barrier()
        pltpu.sync_copy(xes.at[pl.ds(sid * RPT_E, RPT_E)],
                        xep.at[cid, pl.ds(sid * RPT_E, RPT_E),
                               pl.ds(cb * CBA, CBA)])
        plsc.subcore_barrier()


@functools.partial(
    pl.kernel,
    out_type=jax.ShapeDtypeStruct((NC, NV, D), jnp.float32),
    mesh=_mesh,
    scratch_types=[
        pltpu.VMEM((NCHA, KA), jnp.int32),
        pltpu.VMEM((NCHA, KA), jnp.int32),
        pltpu.VMEM((KA, CBA), jnp.float32),
        pltpu.VMEM((KA, CBA), jnp.float32),
        pltpu.VMEM_SHARED((NV, CBA), jnp.float32),
        pltpu.SemaphoreType.DMA,
        pltpu.SemaphoreType.DMA,
    ],
    compiler_params=_sc_params,
)
def _phase_b(xe0, xe1, xe2, xe3, v_hbm, e_hbm, xvp,
             vidx, eidx, stag0, stag1, xvs, sem0, sem1):
    cid = lax.axis_index("c")
    sid = lax.axis_index("s")
    wid = cid * NS + sid

    pltpu.sync_copy(v_hbm.at[wid], vidx)
    pltpu.sync_copy(e_hbm.at[wid], eidx)

    for k, xe in enumerate((xe0, xe1, xe2, xe3)):
        c0 = k * CBA
        _zero_vmem_2d(stag0, KA, CBA)
        for z in range(RPT_V // KA):
            pltpu.sync_copy(stag0, xvs.at[pl.ds(sid * RPT_V + z * KA, KA)])
        plsc.subcore_barrier()

        _gs_pipeline(xe, eidx, xvs, vidx, (stag0, stag1),
                     (sem0, sem1), NCHA)
        plsc.subcore_barrier()

        pltpu.sync_copy(xvs.at[pl.ds(sid * RPT_V, RPT_V)],
                        xvp.at[cid, pl.ds(sid * RPT_V, RPT_V), pl.ds(c0, CBA)])
        plsc.subcore_barrier()


def _merge_body(xep_ref, cep_ref, o_ref):
    ce = jnp.sum(cep_ref[...], axis=0)
    s = 1.0 / jnp.maximum(ce, 1.0)
    xs = (xep_ref[0] + xep_ref[1]) * s[:, None]
    for k in range(NCBA):
        blk = xs[:, k * CBA:(k + 1) * CBA]
        for r in range(REP):
            o_ref[k, r] = blk


def _merge_xe(xep, cep):
    return pl.pallas_call(
        _merge_body,
        out_shape=jax.ShapeDtypeStruct((NCBA, REP, ME, CBA), jnp.float32),
    )(xep, cep)


def _phase_c_body(xv_ref, cv_ref, x0_ref, wt_ref, ab_ref, o_ref):
    a = ab_ref[0, 0]
    b = ab_ref[0, 1]
    xv = xv_ref[0] + xv_ref[1]
    cv = jnp.sum(cv_ref[...], axis=0)
    xv = xv / jnp.maximum(cv, 1.0)[:, None]
    n2 = jnp.sum(xv * xv, axis=1, keepdims=True)
    xc = xv * jnp.where(n2 > 0, lax.rsqrt(n2), 0.0)
    xi = (1.0 - a) * xc + a * x0_ref[...]
    o_ref[...] = (1.0 - b) * xi + b * jnp.dot(
        xi, wt_ref[...], preferred_element_type=jnp.float32)


def _phase_c(xvp, cvp, x0, wt, ab):
    blk = 512
    grid = (N + blk - 1) // blk
    return pl.pallas_call(
        _phase_c_body,
        grid=(grid,),
        in_specs=[
            pl.BlockSpec((NC, blk, D), lambda i: (0, i, 0)),
            pl.BlockSpec((NW, blk), lambda i: (0, i)),
            pl.BlockSpec((blk, D), lambda i: (i, 0)),
            pl.BlockSpec((D, D), lambda i: (0, 0)),
            pl.BlockSpec(memory_space=pltpu.SMEM),
        ],
        out_specs=pl.BlockSpec((blk, D), lambda i: (i, 0)),
        out_shape=jax.ShapeDtypeStruct((N, D), jnp.float32),
    )(xvp, cvp, x0, wt, ab)


def kernel(X, vertex, edges, alpha, beta, X0, W):
    pad = E_PAD - NNZ
    pad_i = jnp.arange(pad, dtype=jnp.int32)
    vflat = jnp.concatenate([vertex.astype(jnp.int32), N + (pad_i % (NV - N))])
    eflat = jnp.concatenate([edges.astype(jnp.int32), M + (pad_i % (ME - M))])
    vp = vflat.reshape(NW, NCHA, KA)
    ep = eflat.reshape(NW, NCHA, KA)
    erep = (eflat + ME * (jnp.arange(E_PAD, dtype=jnp.int32) % REP)
            ).reshape(NW, NCHA, KA)

    xp = jnp.zeros((NV, D), jnp.float32).at[:N].set(X)
    xcols = [xp[:, i * CBA:(i + 1) * CBA] for i in range(NCBA)]

    xep, cep, cvp = _phase_a(*xcols, vp, ep)
    xem = _merge_xe(xep, cep)
    xek = [xem[k].reshape(REP * ME, CBA) for k in range(NCBA)]
    xvp = _phase_b(xek[0], xek[1], xek[2], xek[3], vp, erep)

    ab = jnp.stack([alpha, beta]).reshape(1, 2).astype(jnp.float32)
    return _phase_c(xvp, cvp, X0, W.T, ab)


# speedup vs baseline: 1.1570x; 1.1570x over previous
import functools

import jax
import jax.numpy as jnp
from jax import lax
from jax.experimental import pallas as pl
from jax.experimental.pallas import tpu as pltpu
from jax.experimental.pallas import tpu_sc as plsc

N = 10000
M = 2000
NNZ = 160000
D = 512

NC, NS = 2, 16
NW = NC * NS
EPT = 5120
E_PAD = EPT * NW
ME = 2048
NV = 10240
RPT_E = ME // NS
RPT_V = NV // NS

KA, NCHA = 128, 40
CBA = 128
NCBA = D // CBA
KB, NCHB = 128, 40
CBB = 128
NCBB = D // CBB
KB = 64

_mesh = plsc.VectorSubcoreMesh(core_axis_name="c", subcore_axis_name="s")
_sc_params = pltpu.CompilerParams(needs_layout_passes=False)


def _zero_vmem_2d(ref, rows, cols):
    def body(r, _):
        for l in range(cols // 16):
            ref[r, pl.ds(l * 16, 16)] = jnp.zeros((16,), jnp.float32)
        return 0
    lax.fori_loop(0, rows, body, 0)


def _zero_vmem_1d(ref, n):
    def body(l, _):
        ref[pl.ds(l * 16, 16)] = jnp.zeros((16,), jnp.float32)
        return 0
    lax.fori_loop(0, n // 16, body, 0)


def _gs_pipeline(src, gfn, dst, sfn, stags, sems, nch):
    nbuf = len(stags)
    ngrp = nch // nbuf
    for b in range(nbuf - 1):
        pltpu.async_copy(src.at[gfn(b, b)], stags[b], sems[b])

    def group(g, _):
        j0 = g * nbuf
        for b in range(nbuf):
            j = j0 + b
            pltpu.make_async_copy(src.at[gfn(j, b)], stags[b], sems[b]).wait()
            nb = (b + nbuf - 1) % nbuf
            pltpu.async_copy(src.at[gfn(j + nbuf - 1, nb)], stags[nb], sems[nb])
            pltpu.sync_copy(stags[b], dst.at[sfn(j, b)], add=True)
        return 0

    lax.fori_loop(0, ngrp - 1, group, 0)
    j0 = (ngrp - 1) * nbuf
    for b in range(nbuf):
        j = j0 + b
        pltpu.make_async_copy(src.at[gfn(j, b)], stags[b], sems[b]).wait()
        if j + nbuf - 1 < nch:
            nb = (b + nbuf - 1) % nbuf
            pltpu.async_copy(src.at[gfn(j + nbuf - 1, nb)], stags[nb], sems[nb])
        pltpu.sync_copy(stags[b], dst.at[sfn(j, b)], add=True)


@functools.partial(
    pl.kernel,
    out_type=(
        jax.ShapeDtypeStruct((NC, ME, D), jnp.float32),
        jax.ShapeDtypeStruct((NW, ME), jnp.float32),
        jax.ShapeDtypeStruct((NW, NV), jnp.float32),
    ),
    mesh=_mesh,
    scratch_types=[
        pltpu.VMEM((NCHA, KA), jnp.int32),
        pltpu.VMEM((NCHA, KA), jnp.int32),
        pltpu.VMEM((KA, CBA), jnp.float32),
        pltpu.VMEM((KA, CBA), jnp.float32),
        pltpu.VMEM((KA, CBA), jnp.float32),
        pltpu.VMEM((KA, CBA), jnp.float32),
        pltpu.VMEM((ME,), jnp.float32),
        pltpu.VMEM((NV,), jnp.float32),
        pltpu.VMEM_SHARED((ME, CBA), jnp.float32),
        pltpu.SemaphoreType.DMA,
        pltpu.SemaphoreType.DMA,
        pltpu.SemaphoreType.DMA,
        pltpu.SemaphoreType.DMA,
    ],
    compiler_params=_sc_params,
)
def _phase_a(x0c, x1c, x2c, x3c, v_hbm, e_hbm, xep, cep, cvp,
             vidx, eidx, stag0, stag1, stag2, stag3, ceh, cvh, xes,
             sem0, sem1, sem2, sem3):
    cid = lax.axis_index("c")
    sid = lax.axis_index("s")
    wid = cid * NS + sid

    pltpu.sync_copy(v_hbm.at[wid], vidx)
    pltpu.sync_copy(e_hbm.at[wid], eidx)

    _zero_vmem_1d(ceh, ME)
    _zero_vmem_1d(cvh, NV)

    ones = jnp.ones((16,), jnp.float32)

    def hist(j, _):
        for l in range(KA // 16):
            vv = vidx[j, pl.ds(l * 16, 16)]
            ev = eidx[j, pl.ds(l * 16, 16)]
            plsc.addupdate_scatter(cvh, [vv], ones)
            plsc.addupdate_scatter(ceh, [ev], ones)
        return 0

    lax.fori_loop(0, NCHA, hist, 0)
    pltpu.sync_copy(ceh, cep.at[wid])
    pltpu.sync_copy(cvh, cvp.at[wid])

    for cb, xc in enumerate((x0c, x1c, x2c, x3c)):
        _zero_vmem_2d(stag0, KA, CBA)
        pltpu.sync_copy(stag0, xes.at[pl.ds(sid * RPT_E, RPT_E)])
        plsc.subcore_barrier()

        _gs_pipeline(xc, lambda j, b: vidx.at[j], xes,
                     lambda j, b: eidx.at[j],
                     (stag0, stag1, stag2, stag3),
                     (sem0, sem1, sem2, sem3), NCHA)
        plsc.subcore_barrier()
        pltpu.sync_copy(xes.at[pl.ds(sid * RPT_E, RPT_E)],
                        xep.at[cid, pl.ds(sid * RPT_E, RPT_E),
                               pl.ds(cb * CBA, CBA)])
        plsc.subcore_barrier()


@functools.partial(
    pl.kernel,
    out_type=jax.ShapeDtypeStruct((NC, NV, D), jnp.float32),
    mesh=_mesh,
    scratch_types=[
        pltpu.VMEM((NCHA, KA), jnp.int32),
        pltpu.VMEM((NCHA, KA), jnp.int32),
        pltpu.VMEM((KB, CBA), jnp.float32),
        pltpu.VMEM((KB, CBA), jnp.float32),
        pltpu.VMEM((KB, CBA), jnp.float32),
        pltpu.VMEM((KB, CBA), jnp.float32),
        pltpu.VMEM_SHARED((NV, CBA), jnp.float32),
        pltpu.SemaphoreType.DMA,
        pltpu.SemaphoreType.DMA,
        pltpu.SemaphoreType.DMA,
        pltpu.SemaphoreType.DMA,
    ],
    compiler_params=_sc_params,
)
def _phase_b(xe0, xe1, xe2, xe3, v_hbm, e_hbm, xvp,
             vidx, eidx, stag0, stag1, stag2, stag3, xvs,
             sem0, sem1, sem2, sem3):
    cid = lax.axis_index("c")
    sid = lax.axis_index("s")
    wid = cid * NS + sid

    pltpu.sync_copy(v_hbm.at[wid], vidx)
    pltpu.sync_copy(e_hbm.at[wid], eidx)

    def half(ref):
        return lambda j, b: ref.at[j // 2, pl.ds((b % 2) * KB, KB)]

    for k, xe in enumerate((xe0, xe1, xe2, xe3)):
        c0 = k * CBA
        _zero_vmem_2d(stag0, KB, CBA)
        for z in range(RPT_V // KB):
            pltpu.sync_copy(stag0, xvs.at[pl.ds(sid * RPT_V + z * KB, KB)])
        plsc.subcore_barrier()

        _gs_pipeline(xe, half(eidx), xvs, half(vidx),
                     (stag0, stag1, stag2, stag3),
                     (sem0, sem1, sem2, sem3), NCHA * KA // KB)
        plsc.subcore_barrier()

        pltpu.sync_copy(xvs.at[pl.ds(sid * RPT_V, RPT_V)],
                        xvp.at[cid, pl.ds(sid * RPT_V, RPT_V), pl.ds(c0, CBA)])
        plsc.subcore_barrier()


def _merge_body(xep_ref, cep_ref, o_ref):
    ce = jnp.sum(cep_ref[...], axis=0)
    s = 1.0 / jnp.maximum(ce, 1.0)
    xs = (xep_ref[0] + xep_ref[1]) * s[:, None]
    for k in range(NCBA):
        o_ref[k] = xs[:, k * CBA:(k + 1) * CBA]


def _merge_xe(xep, cep):
    return pl.pallas_call(
        _merge_body,
        out_shape=jax.ShapeDtypeStruct((NCBA, ME, CBA), jnp.float32),
    )(xep, cep)


def _phase_c_body(xv_ref, cv_ref, x0_ref, wt_ref, ab_ref, o_ref):
    a = ab_ref[0, 0]
    b = ab_ref[0, 1]
    xv = xv_ref[0] + xv_ref[1]
    cv = jnp.sum(cv_ref[...], axis=0)
    xv = xv / jnp.maximum(cv, 1.0)[:, None]
    n2 = jnp.sum(xv * xv, axis=1, keepdims=True)
    xc = xv * jnp.where(n2 > 0, lax.rsqrt(n2), 0.0)
    xi = (1.0 - a) * xc + a * x0_ref[...]
    o_ref[...] = (1.0 - b) * xi + b * jnp.dot(
        xi, wt_ref[...], preferred_element_type=jnp.float32)


def _phase_c(xvp, cvp, x0, wt, ab):
    blk = 512
    grid = (N + blk - 1) // blk
    return pl.pallas_call(
        _phase_c_body,
        grid=(grid,),
        in_specs=[
            pl.BlockSpec((NC, blk, D), lambda i: (0, i, 0)),
            pl.BlockSpec((NW, blk), lambda i: (0, i)),
            pl.BlockSpec((blk, D), lambda i: (i, 0)),
            pl.BlockSpec((D, D), lambda i: (0, 0)),
            pl.BlockSpec(memory_space=pltpu.SMEM),
        ],
        out_specs=pl.BlockSpec((blk, D), lambda i: (i, 0)),
        out_shape=jax.ShapeDtypeStruct((N, D), jnp.float32),
    )(xvp, cvp, x0, wt, ab)


def kernel(X, vertex, edges, alpha, beta, X0, W):
    pad = E_PAD - NNZ
    pad_i = jnp.arange(pad, dtype=jnp.int32)
    vflat = jnp.concatenate([vertex.astype(jnp.int32), N + (pad_i % (NV - N))])
    eflat = jnp.concatenate([edges.astype(jnp.int32), M + (pad_i % (ME - M))])
    vp = vflat.reshape(NW, NCHA, KA)
    ep = eflat.reshape(NW, NCHA, KA)
    xp = jnp.zeros((NV, D), jnp.float32).at[:N].set(X)
    xcols = [xp[:, i * CBA:(i + 1) * CBA] for i in range(NCBA)]

    xep, cep, cvp = _phase_a(*xcols, vp, ep)
    xem = _merge_xe(xep, cep)
    xvp = _phase_b(xem[0], xem[1], xem[2], xem[3], vp, ep)

    ab = jnp.stack([alpha, beta]).reshape(1, 2).astype(jnp.float32)
    return _phase_c(xvp, cvp, X0, W.T, ab)
